# baseline (speedup 1.0000x reference)
import jax
import jax.numpy as jnp
from jax.experimental import pallas as pl
from jax.experimental.pallas import tpu as pltpu

N_FEAT = 10000
H = 64
BLK = 200
GRID = N_FEAT // BLK


def _gru_body(x_ref, xw_ref, bias_ref, u_ref, ht_ref, mask_ref, any_ref,
              w1_ref, b1_ref, w2_ref, b2_ref,
              hout_ref, pred_ref, hsum_ref, cnt_ref):
    i = pl.program_id(0)

    @pl.when(i == 0)
    def _init():
        hsum_ref[...] = jnp.zeros_like(hsum_ref)
        cnt_ref[0] = 0.0

    ht = ht_ref[...]
    u3 = u_ref[...]
    u = jnp.sum(u3 * ht[:, None, :], axis=2)
    xg = xw_ref[...] * x_ref[...] + bias_ref[...]

    z = jax.nn.sigmoid(xg[:, 0:H] + u[:, 0:H])
    r = jax.nn.sigmoid(xg[:, H:2 * H] + u[:, H:2 * H])
    htil = jnp.tanh(xg[:, 2 * H:3 * H] + r * u[:, 2 * H:3 * H])
    hg = z * ht + (1.0 - z) * htil

    m = mask_ref[...]
    hm = hg * m
    anym = any_ref[0] > 0.0
    hout_ref[...] = jnp.where(anym, hm, ht)

    hsum_ref[0:1, :] += jnp.sum(hm, axis=0, keepdims=True)
    cnt_ref[0] += jnp.sum(m)

    @pl.when(i == GRID - 1)
    def _fin():
        mean = hsum_ref[0:1, :] / cnt_ref[0]
        h1 = jax.nn.relu(jnp.dot(mean, w1_ref[...].T,
                                 preferred_element_type=jnp.float32) + b1_ref[...])
        logits = jnp.dot(h1, w2_ref[...].T,
                         preferred_element_type=jnp.float32) + b2_ref[...]
        e = jnp.exp(logits - jnp.max(logits, axis=1, keepdims=True))
        p = e / jnp.sum(e, axis=1, keepdims=True)
        pred_ref[...] = jnp.pad(p, ((0, 0), (0, 126)))


def kernel(tim, X, X_hap, mask, Ht, gru_xT_weights, gru_xT_bias, gru_U_weights,
           mlp_W1, mlp_b1, mlp_W2, mlp_b2):
    x2 = X.reshape(N_FEAT, 1)
    xw = gru_xT_weights.reshape(N_FEAT, 3 * H)
    m2 = mask.astype(jnp.float32).reshape(N_FEAT, 1)
    anym = jnp.any(mask).astype(jnp.float32).reshape(1)

    out_shapes = (
        jax.ShapeDtypeStruct((N_FEAT, H), jnp.float32),
        jax.ShapeDtypeStruct((1, 128), jnp.float32),
    )
    grid_spec = pltpu.PrefetchScalarGridSpec(
        num_scalar_prefetch=0,
        grid=(GRID,),
        in_specs=[
            pl.BlockSpec((BLK, 1), lambda i: (i, 0)),
            pl.BlockSpec((BLK, 3 * H), lambda i: (i, 0)),
            pl.BlockSpec((BLK, 3 * H), lambda i: (i, 0)),
            pl.BlockSpec((BLK, 3 * H, H), lambda i: (i, 0, 0)),
            pl.BlockSpec((BLK, H), lambda i: (i, 0)),
            pl.BlockSpec((BLK, 1), lambda i: (i, 0)),
            pl.BlockSpec(memory_space=pltpu.SMEM),
            pl.BlockSpec((H, H), lambda i: (0, 0)),
            pl.BlockSpec((1, H), lambda i: (0, 0)),
            pl.BlockSpec((2, H), lambda i: (0, 0)),
            pl.BlockSpec((1, 2), lambda i: (0, 0)),
        ],
        out_specs=[
            pl.BlockSpec((BLK, H), lambda i: (i, 0)),
            pl.BlockSpec((1, 128), lambda i: (0, 0)),
        ],
        scratch_shapes=[
            pltpu.VMEM((8, H), jnp.float32),
            pltpu.SMEM((1,), jnp.float32),
        ],
    )
    h_curr, pred_pad = pl.pallas_call(
        _gru_body,
        grid_spec=grid_spec,
        out_shape=out_shapes,
    )(x2, xw, gru_xT_bias, gru_U_weights, Ht, m2, anym,
      mlp_W1, mlp_b1.reshape(1, H), mlp_W2, mlp_b2.reshape(1, 2))
    return pred_pad[0, :2], h_curr
